# baseline (speedup 1.0000x reference)
import functools

import jax
import jax.numpy as jnp
from jax import lax
from jax.experimental import pallas as pl
from jax.experimental.pallas import tpu as pltpu
from jax.experimental.pallas import tpu_sc as plsc

_VOCAB = 256
_BATCH = 4096
_HIST = 50
_NC, _NS = 2, 16
_NW = _NC * _NS
_TOTAL = _BATCH * _HIST
_PER_W = _TOTAL // _NW
_CH = 128
_NCHUNK = _PER_W // _CH


def _logits_body(table_ref, w_ref, b_ref, out_ref):
    out_ref[...] = lax.dot_general(
        table_ref[...], w_ref[...], (((1,), (1,)), ((), ())),
        preferred_element_type=jnp.float32) + b_ref[...]


def _compute_logits(table, W, b):
    return pl.pallas_call(
        _logits_body,
        out_shape=jax.ShapeDtypeStruct((_VOCAB, _VOCAB), jnp.float32),
    )(table, W, b.reshape(1, _VOCAB))


_BPW = _BATCH // _NW
_CB = 2
_NCHUNK3 = _BPW // _CB
_NBUF = 2
_HPAD = 56


@functools.partial(
    pl.kernel,
    mesh=plsc.VectorSubcoreMesh(core_axis_name="c", subcore_axis_name="s"),
    out_type=jax.ShapeDtypeStruct((_BATCH, _HPAD, _VOCAB), jnp.float32),
    scratch_types=[
        pltpu.VMEM((_BPW, _HPAD), jnp.int32),
        pltpu.VMEM((_CB, _HPAD, _VOCAB), jnp.float32),
        pltpu.VMEM((_CB, _HPAD, _VOCAB), jnp.float32),
        pltpu.SemaphoreType.DMA,
        pltpu.SemaphoreType.DMA,
        pltpu.SemaphoreType.DMA,
        pltpu.SemaphoreType.DMA,
    ],
)
def _sc_gather(x_hbm, logits_hbm, out_hbm, idx_v, rows0, rows1,
               g0, g1, o0, o1):
    wid = lax.axis_index("s") * _NC + lax.axis_index("c")
    pltpu.sync_copy(x_hbm.at[wid], idx_v)
    bbase = wid * _BPW
    bufs = ((rows0, g0, o0), (rows1, g1, o1))

    def g_start(j, b):
        rows, g, _ = bufs[b]
        for k in range(_CB):
            pltpu.async_copy(logits_hbm.at[idx_v.at[j * _CB + k]],
                             rows.at[k], g)

    def g_wait(j, b):
        rows, g, _ = bufs[b]
        for k in range(_CB):
            pltpu.make_async_copy(logits_hbm.at[idx_v.at[j * _CB + k]],
                                  rows.at[k], g).wait()

    def s_start(j, b):
        rows, _, o = bufs[b]
        pltpu.async_copy(rows, out_hbm.at[pl.ds(bbase + j * _CB, _CB)], o)

    def s_wait(j, b):
        rows, _, o = bufs[b]
        pltpu.make_async_copy(
            rows, out_hbm.at[pl.ds(bbase + j * _CB, _CB)], o).wait()

    for b in range(_NBUF):
        g_start(b, b)

    def body(i, carry):
        j0 = i * _NBUF
        for b in range(_NBUF):
            g_wait(j0 + b, b)
            s_start(j0 + b, b)
        for b in range(_NBUF):
            s_wait(j0 + b, b)
            g_start(j0 + _NBUF + b, b)
        return carry

    lax.fori_loop(0, _NCHUNK3 // _NBUF - 1, body, 0)

    j0 = _NCHUNK3 - _NBUF
    for b in range(_NBUF):
        g_wait(j0 + b, b)
        s_start(j0 + b, b)
    for b in range(_NBUF):
        s_wait(j0 + b, b)


def kernel(x, table, W, b):
    logits = _compute_logits(table, W, b)
    xp = jnp.pad(x.astype(jnp.int32), ((0, 0), (0, _HPAD - _HIST)))
    xf = xp.reshape(_NW, _BPW, _HPAD)
    out = _sc_gather(xf, logits)
    return out[:, :_HIST, :]


# speedup vs baseline: 2.6245x; 2.6245x over previous
import functools

import jax
import jax.numpy as jnp
from jax import lax
from jax.experimental import pallas as pl
from jax.experimental.pallas import tpu as pltpu
from jax.experimental.pallas import tpu_sc as plsc

_VOCAB = 256
_BATCH = 4096
_HIST = 50
_TB = 8


def _logits_body(table_ref, w_ref, b_ref, out_ref):
    out_ref[...] = lax.dot_general(
        table_ref[...], w_ref[...], (((1,), (1,)), ((), ())),
        preferred_element_type=jnp.float32) + b_ref[...]


def _compute_logits(table, W, b):
    return pl.pallas_call(
        _logits_body,
        out_shape=jax.ShapeDtypeStruct((_VOCAB, _VOCAB), jnp.float32),
    )(table, W, b.reshape(1, _VOCAB))


def _onehot_body(x_ref, logits_ref, out_ref):
    lg = logits_ref[...].astype(jnp.bfloat16)
    iota = lax.broadcasted_iota(jnp.int32, (_HIST, _VOCAB), 1)
    for k in range(_TB):
        idxk = x_ref[k]
        ohk = (idxk == iota).astype(jnp.bfloat16)
        out_ref[k] = jnp.dot(ohk, lg, preferred_element_type=jnp.float32)


def _tc_decode(x3, logits):
    return pl.pallas_call(
        _onehot_body,
        grid=(_BATCH // _TB,),
        in_specs=[
            pl.BlockSpec((_TB, _HIST, 1), lambda i: (i, 0, 0)),
            pl.BlockSpec((_VOCAB, _VOCAB), lambda i: (0, 0)),
        ],
        out_specs=pl.BlockSpec((_TB, _HIST, _VOCAB), lambda i: (i, 0, 0)),
        out_shape=jax.ShapeDtypeStruct((_BATCH, _HIST, _VOCAB), jnp.float32),
    )(x3, logits)


def kernel(x, table, W, b):
    logits = _compute_logits(table, W, b)
    x3 = x.astype(jnp.int32).reshape(_BATCH, _HIST, 1)
    return _tc_decode(x3, logits)


# speedup vs baseline: 2.7223x; 1.0373x over previous
import functools

import jax
import jax.numpy as jnp
from jax import lax
from jax.experimental import pallas as pl
from jax.experimental.pallas import tpu as pltpu
from jax.experimental.pallas import tpu_sc as plsc

_VOCAB = 256
_BATCH = 4096
_HIST = 50
_TB = 8


def _logits_body(table_ref, w_ref, b_ref, out_ref):
    out_ref[...] = lax.dot_general(
        table_ref[...], w_ref[...], (((1,), (1,)), ((), ())),
        preferred_element_type=jnp.float32) + b_ref[...]


def _compute_logits(table, W, b):
    return pl.pallas_call(
        _logits_body,
        out_shape=jax.ShapeDtypeStruct((_VOCAB, _VOCAB), jnp.float32),
    )(table, W, b.reshape(1, _VOCAB))


_HPAD = 56


def _onehot_body(x_ref, logits_ref, out_ref):
    lg = logits_ref[...]
    m = _TB * _HPAD
    idx = x_ref[...]
    oh = (idx == lax.broadcasted_iota(
        jnp.int32, (m, _VOCAB), 1)).astype(jnp.bfloat16)
    acc = jnp.dot(oh, lg, preferred_element_type=jnp.float32)
    for k in range(_TB):
        out_ref[k] = acc[k * _HPAD:k * _HPAD + _HIST, :]


def _tc_decode(x2, logits_bf):
    return pl.pallas_call(
        _onehot_body,
        grid=(_BATCH // _TB,),
        in_specs=[
            pl.BlockSpec((_TB * _HPAD, 1), lambda i: (i, 0)),
            pl.BlockSpec((_VOCAB, _VOCAB), lambda i: (0, 0)),
        ],
        out_specs=pl.BlockSpec((_TB, _HIST, _VOCAB), lambda i: (i, 0, 0)),
        out_shape=jax.ShapeDtypeStruct((_BATCH, _HIST, _VOCAB), jnp.float32),
    )(x2, logits_bf)


def kernel(x, table, W, b):
    logits = _compute_logits(table, W, b)
    xp = jnp.pad(x.astype(jnp.int32), ((0, 0), (0, _HPAD - _HIST)),
                 constant_values=-1)
    x2 = xp.reshape(_BATCH * _HPAD, 1)
    return _tc_decode(x2, logits.astype(jnp.bfloat16))


# speedup vs baseline: 3.3811x; 1.2420x over previous
import functools

import jax
import jax.numpy as jnp
from jax import lax
from jax.experimental import pallas as pl
from jax.experimental.pallas import tpu as pltpu
from jax.experimental.pallas import tpu_sc as plsc

_VOCAB = 256
_BATCH = 4096
_HIST = 50
_TB = 8


def _logits_body(table_ref, w_ref, b_ref, out_ref):
    out_ref[...] = lax.dot_general(
        table_ref[...], w_ref[...], (((1,), (1,)), ((), ())),
        preferred_element_type=jnp.float32) + b_ref[...]


def _compute_logits(table, W, b):
    return pl.pallas_call(
        _logits_body,
        out_shape=jax.ShapeDtypeStruct((_VOCAB, _VOCAB), jnp.float32),
    )(table, W, b.reshape(1, _VOCAB))


_HPAD = 56


def _onehot_body(x_ref, logits_ref, out_ref):
    lg = logits_ref[...]
    m = _TB * _HPAD
    idx = x_ref[0]
    oht = (idx == lax.broadcasted_iota(
        jnp.int32, (_VOCAB, m), 0)).astype(jnp.bfloat16)
    acc = lax.dot_general(oht, lg, (((0,), (0,)), ((), ())),
                          preferred_element_type=jnp.float32)
    for k in range(_TB):
        out_ref[k] = acc[k * _HPAD:k * _HPAD + _HIST, :]


def _tc_decode(x2, logits_bf):
    return pl.pallas_call(
        _onehot_body,
        grid=(_BATCH // _TB,),
        in_specs=[
            pl.BlockSpec((1, 1, _TB * _HPAD), lambda i: (i, 0, 0)),
            pl.BlockSpec((_VOCAB, _VOCAB), lambda i: (0, 0)),
        ],
        out_specs=pl.BlockSpec((_TB, _HIST, _VOCAB), lambda i: (i, 0, 0)),
        out_shape=jax.ShapeDtypeStruct((_BATCH, _HIST, _VOCAB), jnp.float32),
    )(x2, logits_bf)


def kernel(x, table, W, b):
    logits = _compute_logits(table, W, b)
    xp = jnp.pad(x.astype(jnp.int32), ((0, 0), (0, _HPAD - _HIST)),
                 constant_values=-1)
    x2 = xp.reshape(_BATCH // _TB, 1, _TB * _HPAD)
    return _tc_decode(x2, logits.astype(jnp.bfloat16))


# speedup vs baseline: 4.6392x; 1.3721x over previous
import functools

import jax
import jax.numpy as jnp
from jax import lax
from jax.experimental import pallas as pl
from jax.experimental.pallas import tpu as pltpu
from jax.experimental.pallas import tpu_sc as plsc

_VOCAB = 256
_BATCH = 4096
_HIST = 50
_TB = 16


def _logits_body(table_ref, w_ref, b_ref, out_ref):
    out_ref[...] = lax.dot_general(
        table_ref[...], w_ref[...], (((1,), (1,)), ((), ())),
        preferred_element_type=jnp.float32) + b_ref[...]


def _compute_logits(table, W, b):
    return pl.pallas_call(
        _logits_body,
        out_shape=jax.ShapeDtypeStruct((_VOCAB, _VOCAB), jnp.float32),
    )(table, W, b.reshape(1, _VOCAB))


_HPAD = 56


def _onehot_body(x_ref, logits_ref, out_ref):
    lg = logits_ref[...]
    m = _TB * _HPAD
    idx = x_ref[0]
    oht = (idx == lax.broadcasted_iota(
        jnp.int32, (_VOCAB, m), 0)).astype(jnp.bfloat16)
    acc = lax.dot_general(oht, lg, (((0,), (0,)), ((), ())),
                          preferred_element_type=jnp.float32)
    for k in range(_TB):
        out_ref[k] = acc[k * _HPAD:k * _HPAD + _HIST, :]


def _tc_decode(x2, logits_bf):
    return pl.pallas_call(
        _onehot_body,
        grid=(_BATCH // _TB,),
        in_specs=[
            pl.BlockSpec((1, 1, _TB * _HPAD), lambda i: (i, 0, 0)),
            pl.BlockSpec((_VOCAB, _VOCAB), lambda i: (0, 0)),
        ],
        out_specs=pl.BlockSpec((_TB, _HIST, _VOCAB), lambda i: (i, 0, 0)),
        out_shape=jax.ShapeDtypeStruct((_BATCH, _HIST, _VOCAB), jnp.float32),
    )(x2, logits_bf)


def kernel(x, table, W, b):
    logits = _compute_logits(table, W, b)
    xp = jnp.pad(x.astype(jnp.int32), ((0, 0), (0, _HPAD - _HIST)),
                 constant_values=-1)
    x2 = xp.reshape(_BATCH // _TB, 1, _TB * _HPAD)
    return _tc_decode(x2, logits.astype(jnp.bfloat16))


# speedup vs baseline: 5.7587x; 1.2413x over previous
import functools

import jax
import jax.numpy as jnp
from jax import lax
from jax.experimental import pallas as pl
from jax.experimental.pallas import tpu as pltpu
from jax.experimental.pallas import tpu_sc as plsc

_VOCAB = 256
_BATCH = 4096
_HIST = 50
_TB = 32


def _logits_body(table_ref, w_ref, b_ref, out_ref):
    out_ref[...] = lax.dot_general(
        table_ref[...], w_ref[...], (((1,), (1,)), ((), ())),
        preferred_element_type=jnp.float32) + b_ref[...]


def _compute_logits(table, W, b):
    return pl.pallas_call(
        _logits_body,
        out_shape=jax.ShapeDtypeStruct((_VOCAB, _VOCAB), jnp.float32),
    )(table, W, b.reshape(1, _VOCAB))


_HPAD = 56


def _onehot_body(x_ref, logits_ref, out_ref):
    lg = logits_ref[...]
    m = _TB * _HPAD
    idx = x_ref[0]
    oht = (idx == lax.broadcasted_iota(
        jnp.int32, (_VOCAB, m), 0)).astype(jnp.bfloat16)
    acc = lax.dot_general(oht, lg, (((0,), (0,)), ((), ())),
                          preferred_element_type=jnp.float32)
    for k in range(_TB):
        out_ref[k] = acc[k * _HPAD:k * _HPAD + _HIST, :]


def _tc_decode(x2, logits_bf):
    return pl.pallas_call(
        _onehot_body,
        grid=(_BATCH // _TB,),
        in_specs=[
            pl.BlockSpec((1, 1, _TB * _HPAD), lambda i: (i, 0, 0)),
            pl.BlockSpec((_VOCAB, _VOCAB), lambda i: (0, 0)),
        ],
        out_specs=pl.BlockSpec((_TB, _HIST, _VOCAB), lambda i: (i, 0, 0)),
        out_shape=jax.ShapeDtypeStruct((_BATCH, _HIST, _VOCAB), jnp.float32),
    )(x2, logits_bf)


def kernel(x, table, W, b):
    logits = _compute_logits(table, W, b)
    xp = jnp.pad(x.astype(jnp.int32), ((0, 0), (0, _HPAD - _HIST)),
                 constant_values=-1)
    x2 = xp.reshape(_BATCH // _TB, 1, _TB * _HPAD)
    return _tc_decode(x2, logits.astype(jnp.bfloat16))


# speedup vs baseline: 6.6246x; 1.1504x over previous
import functools

import jax
import jax.numpy as jnp
from jax import lax
from jax.experimental import pallas as pl
from jax.experimental.pallas import tpu as pltpu
from jax.experimental.pallas import tpu_sc as plsc

_VOCAB = 256
_BATCH = 4096
_HIST = 50
_TB = 64


def _logits_body(table_ref, w_ref, b_ref, out_ref):
    out_ref[...] = lax.dot_general(
        table_ref[...], w_ref[...], (((1,), (1,)), ((), ())),
        preferred_element_type=jnp.float32) + b_ref[...]


def _compute_logits(table, W, b):
    return pl.pallas_call(
        _logits_body,
        out_shape=jax.ShapeDtypeStruct((_VOCAB, _VOCAB), jnp.float32),
    )(table, W, b.reshape(1, _VOCAB))


_HPAD = 56


def _onehot_body(x_ref, logits_ref, out_ref):
    lg = logits_ref[...]
    m = _TB * _HPAD
    idx = x_ref[0]
    oht = (idx == lax.broadcasted_iota(
        jnp.int32, (_VOCAB, m), 0)).astype(jnp.bfloat16)
    acc = lax.dot_general(oht, lg, (((0,), (0,)), ((), ())),
                          preferred_element_type=jnp.float32)
    for k in range(_TB):
        out_ref[k] = acc[k * _HPAD:k * _HPAD + _HIST, :]


def _tc_decode(x2, logits_bf):
    return pl.pallas_call(
        _onehot_body,
        grid=(_BATCH // _TB,),
        in_specs=[
            pl.BlockSpec((1, 1, _TB * _HPAD), lambda i: (i, 0, 0)),
            pl.BlockSpec((_VOCAB, _VOCAB), lambda i: (0, 0)),
        ],
        out_specs=pl.BlockSpec((_TB, _HIST, _VOCAB), lambda i: (i, 0, 0)),
        out_shape=jax.ShapeDtypeStruct((_BATCH, _HIST, _VOCAB), jnp.float32),
    )(x2, logits_bf)


def kernel(x, table, W, b):
    logits = _compute_logits(table, W, b)
    xp = jnp.pad(x.astype(jnp.int32), ((0, 0), (0, _HPAD - _HIST)),
                 constant_values=-1)
    x2 = xp.reshape(_BATCH // _TB, 1, _TB * _HPAD)
    return _tc_decode(x2, logits.astype(jnp.bfloat16))


# speedup vs baseline: 6.9606x; 1.0507x over previous
import functools

import jax
import jax.numpy as jnp
from jax import lax
from jax.experimental import pallas as pl
from jax.experimental.pallas import tpu as pltpu
from jax.experimental.pallas import tpu_sc as plsc

_VOCAB = 256
_BATCH = 4096
_HIST = 50
_TB = 128


def _logits_body(table_ref, w_ref, b_ref, out_ref):
    out_ref[...] = lax.dot_general(
        table_ref[...], w_ref[...], (((1,), (1,)), ((), ())),
        preferred_element_type=jnp.float32) + b_ref[...]


def _compute_logits(table, W, b):
    return pl.pallas_call(
        _logits_body,
        out_shape=jax.ShapeDtypeStruct((_VOCAB, _VOCAB), jnp.float32),
    )(table, W, b.reshape(1, _VOCAB))


_HPAD = 56


def _onehot_body(x_ref, logits_ref, out_ref):
    lg = logits_ref[...]
    m = _TB * _HPAD
    idx = x_ref[0]
    oht = (idx == lax.broadcasted_iota(
        jnp.int32, (_VOCAB, m), 0)).astype(jnp.bfloat16)
    acc = lax.dot_general(oht, lg, (((0,), (0,)), ((), ())),
                          preferred_element_type=jnp.float32)
    for k in range(_TB):
        out_ref[k] = acc[k * _HPAD:k * _HPAD + _HIST, :]


def _tc_decode(x2, logits_bf):
    return pl.pallas_call(
        _onehot_body,
        grid=(_BATCH // _TB,),
        in_specs=[
            pl.BlockSpec((1, 1, _TB * _HPAD), lambda i: (i, 0, 0)),
            pl.BlockSpec((_VOCAB, _VOCAB), lambda i: (0, 0)),
        ],
        out_specs=pl.BlockSpec((_TB, _HIST, _VOCAB), lambda i: (i, 0, 0)),
        out_shape=jax.ShapeDtypeStruct((_BATCH, _HIST, _VOCAB), jnp.float32),
    )(x2, logits_bf)


def kernel(x, table, W, b):
    logits = _compute_logits(table, W, b)
    xp = jnp.pad(x.astype(jnp.int32), ((0, 0), (0, _HPAD - _HIST)),
                 constant_values=-1)
    x2 = xp.reshape(_BATCH // _TB, 1, _TB * _HPAD)
    return _tc_decode(x2, logits.astype(jnp.bfloat16))
